# baseline (speedup 1.0000x reference)
import functools

import jax
import jax.numpy as jnp
from jax import lax
from jax.experimental import pallas as pl
from jax.experimental.pallas import tpu as pltpu
from jax.experimental.pallas import tpu_sc as plsc

D_MODEL = 64
NUM_WORKERS = 32
SUB = 128
CHUNK = 1024
NSUB = CHUNK // SUB


@functools.lru_cache(maxsize=None)
def _build(n_rows: int):
    b_per_w = n_rows // NUM_WORKERS
    n_chunks = b_per_w // CHUNK
    mesh = plsc.VectorSubcoreMesh(core_axis_name="c", subcore_axis_name="s")

    @functools.partial(
        pl.kernel,
        mesh=mesh,
        out_type=jax.ShapeDtypeStruct((n_rows, D_MODEL), jnp.float32),
        scratch_types=[
            pltpu.VMEM((NSUB, SUB), jnp.int32),
            pltpu.VMEM((CHUNK, D_MODEL), jnp.float32),
            pltpu.SemaphoreType.DMA,
        ],
        compiler_params=pltpu.CompilerParams(use_tc_tiling_on_sc=False),
    )
    def emb_kernel(idx_hbm, table_hbm, out_hbm, idx_v, rows_v, sem):
        num_cores = 2
        wid = lax.axis_index("s") * num_cores + lax.axis_index("c")
        base = wid * b_per_w

        def body(i, _):
            row0 = base + i * CHUNK
            pltpu.sync_copy(
                idx_hbm.at[pl.ds(pl.multiple_of(row0 // SUB, 8), NSUB)], idx_v
            )
            copies = []
            for j in range(NSUB):
                copies.append(
                    pltpu.async_copy(
                        table_hbm.at[idx_v.at[j]],
                        rows_v.at[pl.ds(j * SUB, SUB)],
                        sem,
                    )
                )
            for c in copies:
                c.wait()
            pltpu.sync_copy(rows_v, out_hbm.at[pl.ds(row0, CHUNK)])
            return 0

        lax.fori_loop(0, n_chunks, body, 0)

    return emb_kernel


def kernel(input_ids, embedding_weight):
    B, T = input_ids.shape
    n_rows = B * T
    ids = input_ids.astype(jnp.int32).reshape(n_rows // SUB, SUB)
    out = _build(n_rows)(ids, embedding_weight)
    return out.reshape(B, T, D_MODEL)


# speedup vs baseline: 1.0053x; 1.0053x over previous
import functools

import jax
import jax.numpy as jnp
from jax import lax
from jax.experimental import pallas as pl
from jax.experimental.pallas import tpu as pltpu
from jax.experimental.pallas import tpu_sc as plsc

D_MODEL = 64
NUM_WORKERS = 32
SUB = 128
CHUNK = 512
NSUB = CHUNK // SUB
NBUF = 2


@functools.lru_cache(maxsize=None)
def _build(n_rows: int):
    b_per_w = n_rows // NUM_WORKERS
    n_chunks = b_per_w // CHUNK
    n_outer = n_chunks // NBUF
    mesh = plsc.VectorSubcoreMesh(core_axis_name="c", subcore_axis_name="s")

    @functools.partial(
        pl.kernel,
        mesh=mesh,
        out_type=jax.ShapeDtypeStruct((n_rows, D_MODEL), jnp.float32),
        scratch_types=[
            pltpu.VMEM((NBUF, NSUB, SUB), jnp.int32),
            pltpu.VMEM((NBUF, CHUNK, D_MODEL), jnp.float32),
            [pltpu.SemaphoreType.DMA] * NBUF,
            [pltpu.SemaphoreType.DMA] * NBUF,
        ],
        compiler_params=pltpu.CompilerParams(use_tc_tiling_on_sc=False),
    )
    def emb_kernel(idx_hbm, table_hbm, out_hbm, idx_v, rows_v, gsems, wsems):
        num_cores = 2
        wid = lax.axis_index("s") * num_cores + lax.axis_index("c")
        base = wid * b_per_w

        def out_slice(row0):
            return out_hbm.at[pl.ds(row0, CHUNK)]

        def body(io, _):
            for b in range(NBUF):
                i = io * NBUF + b
                row0 = base + i * CHUNK
                @pl.when(io > 0)
                def _():
                    pltpu.make_async_copy(
                        rows_v.at[b], out_slice(row0), wsems[b]
                    ).wait()

                pltpu.sync_copy(idx_hbm.at[row0 // CHUNK], idx_v.at[b])
                copies = []
                for j in range(NSUB):
                    copies.append(
                        pltpu.async_copy(
                            table_hbm.at[idx_v.at[b].at[j]],
                            rows_v.at[b].at[pl.ds(j * SUB, SUB)],
                            gsems[b],
                        )
                    )
                for c in copies:
                    c.wait()
                pltpu.async_copy(rows_v.at[b], out_slice(row0), wsems[b])
            return 0

        lax.fori_loop(0, n_outer, body, 0)
        for b in range(NBUF):
            row0 = base + (n_chunks - NBUF + b) * CHUNK
            pltpu.make_async_copy(rows_v.at[b], out_slice(row0), wsems[b]).wait()

    return emb_kernel


def kernel(input_ids, embedding_weight):
    B, T = input_ids.shape
    n_rows = B * T
    ids = input_ids.astype(jnp.int32).reshape(n_rows // CHUNK, NSUB, SUB)
    out = _build(n_rows)(ids, embedding_weight)
    return out.reshape(B, T, D_MODEL)


# speedup vs baseline: 1.0106x; 1.0053x over previous
import functools

import jax
import jax.numpy as jnp
from jax import lax
from jax.experimental import pallas as pl
from jax.experimental.pallas import tpu as pltpu
from jax.experimental.pallas import tpu_sc as plsc

D_MODEL = 64
NUM_WORKERS = 32
CHUNK_B = 4
NBUF = 2


@functools.lru_cache(maxsize=None)
def _build(B: int, T: int):
    b_per_w = B // NUM_WORKERS
    n_chunks = b_per_w // CHUNK_B
    n_outer = n_chunks // NBUF
    mesh = plsc.VectorSubcoreMesh(core_axis_name="c", subcore_axis_name="s")

    @functools.partial(
        pl.kernel,
        mesh=mesh,
        out_type=jax.ShapeDtypeStruct((B, T, D_MODEL), jnp.float32),
        scratch_types=[
            pltpu.VMEM((NBUF, CHUNK_B, T), jnp.int32),
            pltpu.VMEM((NBUF, CHUNK_B, T, D_MODEL), jnp.float32),
            [pltpu.SemaphoreType.DMA] * NBUF,
            [pltpu.SemaphoreType.DMA] * NBUF,
        ],
        compiler_params=pltpu.CompilerParams(use_tc_tiling_on_sc=False),
    )
    def emb_kernel(idx_hbm, table_hbm, out_hbm, idx_v, rows_v, gsems, wsems):
        num_cores = 2
        wid = lax.axis_index("s") * num_cores + lax.axis_index("c")
        base = wid * b_per_w

        def body(io, _):
            for b in range(NBUF):
                i = io * NBUF + b
                b0 = base + i * CHUNK_B
                @pl.when(io > 0)
                def _():
                    pltpu.make_async_copy(
                        rows_v.at[b], out_hbm.at[pl.ds(b0, CHUNK_B)], wsems[b]
                    ).wait()

                pltpu.sync_copy(idx_hbm.at[pl.ds(b0, CHUNK_B)], idx_v.at[b])
                copies = []
                for j in range(CHUNK_B):
                    copies.append(
                        pltpu.async_copy(
                            table_hbm.at[idx_v.at[b].at[j]],
                            rows_v.at[b].at[j],
                            gsems[b],
                        )
                    )
                for c in copies:
                    c.wait()
                pltpu.async_copy(
                    rows_v.at[b], out_hbm.at[pl.ds(b0, CHUNK_B)], wsems[b]
                )
            return 0

        lax.fori_loop(0, n_outer, body, 0)
        for b in range(NBUF):
            b0 = base + (n_chunks - NBUF + b) * CHUNK_B
            pltpu.make_async_copy(
                rows_v.at[b], out_hbm.at[pl.ds(b0, CHUNK_B)], wsems[b]
            ).wait()

    return emb_kernel


def kernel(input_ids, embedding_weight):
    B, T = input_ids.shape
    ids = input_ids.astype(jnp.int32)
    return _build(B, T)(ids, embedding_weight)
